# Initial kernel scaffold; baseline (speedup 1.0000x reference)
import jax
import jax.numpy as jnp
from jax.experimental import pallas as pl


def kernel(h, r_feat, edge_feat, e_w, params, edge_index):
    raise NotImplementedError("write your pallas kernel here")


# speedup vs baseline: 3.4725x; 3.4725x over previous
import functools
import math

import jax
import jax.numpy as jnp
import numpy as np
from jax import lax
from jax.experimental import pallas as pl
from jax.experimental.pallas import tpu as pltpu
from jax.experimental.pallas import tpu_sc as plsc

_N = 10000
_E = 320000
_D = 128
_H = 16
_HD = 8
_EF = 16
_RF = 16
_HID = 128

_NC = 2
_NS = 16
_NW = _NC * _NS
_CHUNK = 128
_CH = 80
_NHALF = 5120
_NOUT = 2 * _NHALF
_TROWS = 8
_ACCROWS = _NHALF + _NS * _TROWS
_CROWS = _NHALF // _NS
_SROWS = 64
_LCAP = 12288
_BCH = 64
_EPAD = _NW * _CH * _CHUNK
_CHT = (_EPAD // _CHUNK) // _NS

_NBLK = 1000
_EBLK = 512

_PREC = jax.lax.Precision.HIGHEST


def _mm(a, b):
    return jax.lax.dot_general(
        a, b, (((1,), (0,)), ((), ())),
        preferred_element_type=jnp.float32, precision=_PREC)


def _ln_relu(h1, g, bg):
    mu = jnp.mean(h1, -1, keepdims=True)
    var = jnp.mean((h1 - mu) * (h1 - mu), -1, keepdims=True)
    hn = (h1 - mu) / jnp.sqrt(var + 1e-5) * g + bg
    return jnp.maximum(hn, 0.0)


def _q_body(h_ref, w1, b1, g, bg, w2, b2, t1_ref):
    hb = h_ref[...]
    a = _ln_relu(_mm(hb, w1[...]) + b1[...], g[...], bg[...])
    q = _mm(a, w2[...]) + b2[...]
    t1_ref[:, :_D] = hb
    t1_ref[:, _D:] = q


def _run_q(h, p):
    full = lambda i: (0, 0)
    return pl.pallas_call(
        _q_body,
        grid=(_N // _NBLK,),
        in_specs=[
            pl.BlockSpec((_NBLK, _D), lambda i: (i, 0)),
            pl.BlockSpec((_D, _HID), full),
            pl.BlockSpec((1, _HID), full),
            pl.BlockSpec((1, _HID), full),
            pl.BlockSpec((1, _HID), full),
            pl.BlockSpec((_HID, _D), full),
            pl.BlockSpec((1, _D), full),
        ],
        out_specs=pl.BlockSpec((_NBLK, 2 * _D), lambda i: (i, 0)),
        out_shape=jax.ShapeDtypeStruct((_N, 2 * _D), jnp.float32),
    )(h, p['W1'], p['b1'].reshape(1, -1), p['g'].reshape(1, -1),
      p['bg'].reshape(1, -1), p['W2'], p['b2'].reshape(1, -1))


def _gather_body(t1, t2, dstp, srcp, g1, g2, idx_v, rows1, rows2, sem):
    wid = lax.axis_index("s") * _NC + lax.axis_index("c")

    def body(j, c):
        base = (wid * _CH + j) * _CHUNK
        pltpu.sync_copy(dstp.at[pl.ds(base, _CHUNK)], idx_v)
        pltpu.async_copy(t1.at[idx_v], rows1, sem).wait()
        pltpu.sync_copy(rows1, g1.at[pl.ds(base, _CHUNK)])
        pltpu.sync_copy(srcp.at[pl.ds(base, _CHUNK)], idx_v)
        pltpu.async_copy(t2.at[idx_v], rows2, sem).wait()
        pltpu.sync_copy(rows2, g2.at[pl.ds(base, _CHUNK)])
        return c

    lax.fori_loop(0, _CH, body, 0)


def _gather_call():
  return functools.partial(
    pl.kernel,
    mesh=plsc.VectorSubcoreMesh(core_axis_name="c", subcore_axis_name="s", num_cores=_NC, num_subcores=_NS),
    out_type=[
        jax.ShapeDtypeStruct((_EPAD, 2 * _D), jnp.float32),
        jax.ShapeDtypeStruct((_EPAD, _D), jnp.float32),
    ],
    scratch_types=[
        pltpu.VMEM((_CHUNK,), jnp.int32),
        pltpu.VMEM((_CHUNK, 2 * _D), jnp.float32),
        pltpu.VMEM((_CHUNK, _D), jnp.float32),
        pltpu.SemaphoreType.DMA,
    ],
  )


def _edge_body(ef, rf, ew, g1, g2,
               k_ef, k_rf, k_hi, k_hj, k_b1, k_g, k_bg, k_w2, k_b2,
               v_ef, v_rf, v_hi, v_hj, v_b1, v_g, v_bg, v_w2, v_b2,
               p16, p16t, m_ref, x_ref):
    hi = g1[:, :_D]
    qd = g1[:, _D:]
    hj = g2[...]
    efb = ef[...]
    rfb = rf[...]

    pre_k = (_mm(efb, k_ef[...]) + _mm(rfb, k_rf[...]) +
             _mm(hi, k_hi[...]) + _mm(hj, k_hj[...]) + k_b1[...])
    k = _mm(_ln_relu(pre_k, k_g[...], k_bg[...]), k_w2[...]) + k_b2[...]

    pre_v = (_mm(efb, v_ef[...]) + _mm(rfb, v_rf[...]) +
             _mm(hi, v_hi[...]) + _mm(hj, v_hj[...]) + v_b1[...])
    v = (_mm(_ln_relu(pre_v, v_g[...], v_bg[...]), v_w2[...]) + v_b2[...]) * ew[...]

    s = _mm(qd * k, p16[...]) * (1.0 / math.sqrt(_HD))
    pos = pl.program_id(0) * _EBLK + lax.broadcasted_iota(jnp.int32, (_EBLK, _H), 0)
    ex = jnp.where(pos < _E, jnp.exp(s), 0.0)
    m_ref[...] = _mm(ex, p16t[...]) * v
    x_ref[:, :_H] = ex
    x_ref[:, _H:] = jnp.zeros((_EBLK, _D - _H), jnp.float32)


def _run_edge(efp, rfp, ewp, g1, g2, pk, pv, p16, p16t):
    full = lambda i: (0, 0)
    eb = lambda i: (i, 0)

    def wspecs():
        return [
            pl.BlockSpec((_EF, _HID), full), pl.BlockSpec((_RF, _HID), full),
            pl.BlockSpec((_D, _HID), full), pl.BlockSpec((_D, _HID), full),
            pl.BlockSpec((1, _HID), full), pl.BlockSpec((1, _HID), full),
            pl.BlockSpec((1, _HID), full), pl.BlockSpec((_HID, _D), full),
            pl.BlockSpec((1, _D), full),
        ]

    def wargs(p):
        return [p['W1'][:_EF], p['W1'][_EF:_EF + _RF],
                p['W1'][_EF + _RF:_EF + _RF + _D], p['W1'][_EF + _RF + _D:],
                p['b1'].reshape(1, -1), p['g'].reshape(1, -1),
                p['bg'].reshape(1, -1), p['W2'], p['b2'].reshape(1, -1)]

    return pl.pallas_call(
        _edge_body,
        grid=(_EPAD // _EBLK,),
        in_specs=[
            pl.BlockSpec((_EBLK, _EF), eb),
            pl.BlockSpec((_EBLK, _RF), eb),
            pl.BlockSpec((_EBLK, 1), eb),
            pl.BlockSpec((_EBLK, 2 * _D), eb),
            pl.BlockSpec((_EBLK, _D), eb),
        ] + wspecs() + wspecs() + [
            pl.BlockSpec((_D, _H), full),
            pl.BlockSpec((_H, _D), full),
        ],
        out_specs=[pl.BlockSpec((_EBLK, _D), eb),
                   pl.BlockSpec((_EBLK, _D), eb)],
        out_shape=[jax.ShapeDtypeStruct((_EPAD, _D), jnp.float32),
                   jax.ShapeDtypeStruct((_EPAD, _D), jnp.float32)],
    )(efp, rfp, ewp, g1, g2, *wargs(pk), *wargs(pv), p16, p16t)


def _rank_body(d_ref, lstrict, offs_ref, base_ref):
    @pl.when(pl.program_id(0) == 0)
    def _():
        base_ref[...] = jnp.zeros((1, _NW), jnp.float32)

    d = d_ref[...]
    tile = d // _CROWS
    tcols = lax.broadcasted_iota(jnp.int32, (_EBLK, _NW), 1)
    onehot = (tile == tcols).astype(jnp.float32)
    prefix = _mm(lstrict[...], onehot)
    base = base_ref[...]
    slotf = jnp.sum(onehot * (prefix + base), -1, keepdims=True)
    slot = jnp.minimum(slotf.astype(jnp.int32), _LCAP - 1)
    offs_ref[...] = tile * _LCAP + slot
    base_ref[...] = base + jnp.sum(onehot, 0, keepdims=True)


def _run_rank(dstp2, lstrict):
    return pl.pallas_call(
        _rank_body,
        grid=(_EPAD // _EBLK,),
        in_specs=[
            pl.BlockSpec((_EBLK, 1), lambda i: (i, 0)),
            pl.BlockSpec((_EBLK, _EBLK), lambda i: (0, 0)),
        ],
        out_specs=pl.BlockSpec((_EBLK, 1), lambda i: (i, 0)),
        out_shape=jax.ShapeDtypeStruct((_EPAD, 1), jnp.int32),
        scratch_shapes=[pltpu.VMEM((1, _NW), jnp.float32)],
    )(dstp2, lstrict)


def _scatter_body(m_hbm, x_hbm, dstp, offs_hbm, fillpos, fillloc, zm, zx,
                  outm, outx, idx_v, off_v, poslist, loclist, mrows, xrows,
                  accm, accx, sem):
    cid = lax.axis_index("c")
    sid = lax.axis_index("s")
    wid = cid * _NS + sid
    lo = wid * _CROWS
    lane = lax.iota(jnp.int32, 16)

    pltpu.sync_copy(fillpos, poslist)
    pltpu.sync_copy(fillloc, loclist)
    pltpu.sync_copy(zm, accm)
    pltpu.sync_copy(zx, accx)

    def scan(j, dummy):
        base = j * _CHUNK
        pltpu.sync_copy(dstp.at[pl.ds(base, _CHUNK)], idx_v)
        pltpu.sync_copy(offs_hbm.at[pl.ds(base, _CHUNK)], off_v)
        for k in range(_CHUNK // 16):
            ix = idx_v[pl.ds(k * 16, 16)]
            lof = off_v[pl.ds(k * 16, 16)] - wid * _LCAP
            ok = jnp.logical_and(lof >= 0, lof < _LCAP)
            lof2 = jnp.where(ok, lof, _LCAP + 16)
            pos16 = base + k * 16 + lane
            plsc.store_scatter(poslist, [lof2], pos16)
            plsc.store_scatter(loclist, [lof2], ix - lo)
        return dummy

    lax.fori_loop(0, _EPAD // _CHUNK, scan, 0)

    def batch(b, c):
        bb = b * _BCH
        pltpu.async_copy(m_hbm.at[poslist.at[pl.ds(bb, _BCH)]], mrows,
                         sem).wait()
        pltpu.async_copy(x_hbm.at[poslist.at[pl.ds(bb, _BCH)]], xrows,
                         sem).wait()

        def acc_grp(gi, c2):
            e0 = gi * 16
            ev = e0 + lane
            loc16 = loclist[pl.ds(bb + e0, 16)]
            for g in range(_D):
                gv = jnp.full((16,), g, jnp.int32)
                vals = plsc.load_gather(mrows, [ev, gv])
                plsc.addupdate_scatter(accm, [loc16, gv], vals)
            for g in range(_H):
                gv = jnp.full((16,), g, jnp.int32)
                vals = plsc.load_gather(xrows, [ev, gv])
                plsc.addupdate_scatter(accx, [loc16, gv], vals)
            return c2

        lax.fori_loop(0, _BCH // 16, acc_grp, 0)
        return c

    lax.fori_loop(0, _LCAP // _BCH, batch, 0)

    pltpu.sync_copy(accm.at[pl.ds(0, _CROWS)], outm.at[pl.ds(lo, _CROWS)])
    pltpu.sync_copy(accx.at[pl.ds(0, _CROWS)], outx.at[pl.ds(lo, _CROWS)])


def _scatter_call():
  return functools.partial(
    pl.kernel,
    mesh=plsc.VectorSubcoreMesh(core_axis_name="c", subcore_axis_name="s", num_cores=_NC, num_subcores=_NS),
    compiler_params=pltpu.CompilerParams(needs_layout_passes=False),
    out_type=[jax.ShapeDtypeStruct((_NOUT, _D), jnp.float32),
              jax.ShapeDtypeStruct((_NOUT, _H), jnp.float32)],
    scratch_types=[
        pltpu.VMEM((_CHUNK,), jnp.int32),
        pltpu.VMEM((_CHUNK,), jnp.int32),
        pltpu.VMEM((_LCAP + 32,), jnp.int32),
        pltpu.VMEM((_LCAP + 32,), jnp.int32),
        pltpu.VMEM((_BCH, _D), jnp.float32),
        pltpu.VMEM((_BCH, _D), jnp.float32),
        pltpu.VMEM((_CROWS + 8, _D), jnp.float32),
        pltpu.VMEM((_CROWS + 8, _H), jnp.float32),
        pltpu.SemaphoreType.DMA,
    ],
  )


def _topo_body(m0, x0, h_ref, w1o, w1h, b1, g, bg, w2, b2, p16t, out_ref):
    den = x0[:, :_H] + 1e-16
    r128 = _mm(1.0 / den, p16t[...])
    o = m0[...] * r128
    hb = h_ref[...]
    pre = _mm(o, w1o[...]) + _mm(hb, w1h[...]) + b1[...]
    out_ref[...] = _mm(_ln_relu(pre, g[...], bg[...]), w2[...]) + b2[...]


def _run_topo(pm, px, h, p, p16t):
    full = lambda i: (0, 0)
    nb = lambda i: (i, 0)
    return pl.pallas_call(
        _topo_body,
        grid=(_N // _NBLK,),
        in_specs=[
            pl.BlockSpec((_NBLK, _D), nb),
            pl.BlockSpec((_NBLK, _D), nb),
            pl.BlockSpec((_NBLK, _D), nb),
            pl.BlockSpec((_D, _HID), full), pl.BlockSpec((_D, _HID), full),
            pl.BlockSpec((1, _HID), full), pl.BlockSpec((1, _HID), full),
            pl.BlockSpec((1, _HID), full), pl.BlockSpec((_HID, _HID), full),
            pl.BlockSpec((1, _HID), full), pl.BlockSpec((_H, _D), full),
        ],
        out_specs=pl.BlockSpec((_NBLK, _HID), nb),
        out_shape=jax.ShapeDtypeStruct((_N, _HID), jnp.float32),
    )(pm, px, h,
      p['W1'][:_D], p['W1'][_D:], p['b1'].reshape(1, -1),
      p['g'].reshape(1, -1), p['bg'].reshape(1, -1), p['W2'],
      p['b2'].reshape(1, -1), p16t)


def kernel(h, r_feat, edge_feat, e_w, params, edge_index):
    src = edge_index[0]
    dst = edge_index[1]
    pad = _EPAD - _E
    dstp = jnp.pad(dst, (0, pad))
    srcp = jnp.pad(src, (0, pad))
    efp = jnp.pad(edge_feat, ((0, pad), (0, 0)))
    rfp = jnp.pad(r_feat, ((0, pad), (0, 0)))
    ewp = jnp.pad(e_w, (0, pad)).reshape(_EPAD, 1)

    p16 = jnp.asarray(np.repeat(np.eye(_H, dtype=np.float32), _HD, axis=0))
    p16t = jnp.asarray(p16.T)

    t1 = _run_q(h, params['xq'])
    g1, g2 = _gather_call()(_gather_body)(t1, h, dstp, srcp)
    m, x = _run_edge(efp, rfp, ewp, g1, g2, params['xk'], params['xv'],
                     p16, p16t)
    fillpos = jnp.full((_LCAP + 32,), _E, jnp.int32)
    fillloc = jnp.full((_LCAP + 32,), _CROWS, jnp.int32)
    zm = jnp.zeros((_CROWS + 8, _D), jnp.float32)
    zx = jnp.zeros((_CROWS + 8, _H), jnp.float32)
    dsts = jnp.pad(dst, (0, pad), constant_values=_N + 16)
    lstrict = jnp.asarray(np.tril(np.ones((_EBLK, _EBLK), np.float32), -1))
    offs = _run_rank(dsts.reshape(_EPAD, 1), lstrict).reshape(_EPAD)
    pm, px = _scatter_call()(_scatter_body)(m, x, dsts, offs, fillpos, fillloc,
                                            zm, zx)
    return _run_topo(pm, px, h, params['topo'], p16t)


# speedup vs baseline: 4.1557x; 1.1967x over previous
import functools
import math

import jax
import jax.numpy as jnp
import numpy as np
from jax import lax
from jax.experimental import pallas as pl
from jax.experimental.pallas import tpu as pltpu
from jax.experimental.pallas import tpu_sc as plsc

_N = 10000
_E = 320000
_D = 128
_H = 16
_HD = 8
_EF = 16
_RF = 16
_HID = 128

_NC = 2
_NS = 16
_NW = _NC * _NS
_CHUNK = 128
_CH = 80
_NHALF = 5120
_NOUT = 2 * _NHALF
_TROWS = 8
_ACCROWS = _NHALF + _NS * _TROWS
_CROWS = _NHALF // _NS
_SROWS = 64
_LCAP = 12288
_BCH = 64
_IBLK = 1024
_EPAD = _NW * _CH * _CHUNK
_CHT = (_EPAD // _CHUNK) // _NS

_NBLK = 1000
_EBLK = 512

_PREC = jax.lax.Precision.HIGHEST


def _mm(a, b):
    return jax.lax.dot_general(
        a, b, (((1,), (0,)), ((), ())),
        preferred_element_type=jnp.float32, precision=_PREC)


def _ln_relu(h1, g, bg):
    mu = jnp.mean(h1, -1, keepdims=True)
    var = jnp.mean((h1 - mu) * (h1 - mu), -1, keepdims=True)
    hn = (h1 - mu) / jnp.sqrt(var + 1e-5) * g + bg
    return jnp.maximum(hn, 0.0)


def _q_body(h_ref, w1, b1, g, bg, w2, b2, t1_ref):
    hb = h_ref[...]
    a = _ln_relu(_mm(hb, w1[...]) + b1[...], g[...], bg[...])
    q = _mm(a, w2[...]) + b2[...]
    t1_ref[:, :_D] = hb
    t1_ref[:, _D:] = q


def _run_q(h, p):
    full = lambda i: (0, 0)
    return pl.pallas_call(
        _q_body,
        grid=(_N // _NBLK,),
        in_specs=[
            pl.BlockSpec((_NBLK, _D), lambda i: (i, 0)),
            pl.BlockSpec((_D, _HID), full),
            pl.BlockSpec((1, _HID), full),
            pl.BlockSpec((1, _HID), full),
            pl.BlockSpec((1, _HID), full),
            pl.BlockSpec((_HID, _D), full),
            pl.BlockSpec((1, _D), full),
        ],
        out_specs=pl.BlockSpec((_NBLK, 2 * _D), lambda i: (i, 0)),
        out_shape=jax.ShapeDtypeStruct((_N, 2 * _D), jnp.float32),
    )(h, p['W1'], p['b1'].reshape(1, -1), p['g'].reshape(1, -1),
      p['bg'].reshape(1, -1), p['W2'], p['b2'].reshape(1, -1))


def _gather_body(t1, t2, dstp, srcp, g1, g2, dst_v, src_v, rows1, rows2,
                 sem, sem2):
    wid = lax.axis_index("s") * _NC + lax.axis_index("c")

    def body(j, c):
        blk = (wid * (_CH // 8) + j) * _IBLK
        pltpu.sync_copy(dstp.at[pl.ds(blk, _IBLK)], dst_v)
        pltpu.sync_copy(srcp.at[pl.ds(blk, _IBLK)], src_v)
        for k in range(_IBLK // _CHUNK):
            base = blk + k * _CHUNK
            d1 = pltpu.async_copy(t1.at[dst_v.at[pl.ds(k * _CHUNK, _CHUNK)]],
                                  rows1, sem)
            d2 = pltpu.async_copy(t2.at[src_v.at[pl.ds(k * _CHUNK, _CHUNK)]],
                                  rows2, sem2)
            d1.wait()
            pltpu.sync_copy(rows1, g1.at[pl.ds(base, _CHUNK)])
            d2.wait()
            pltpu.sync_copy(rows2, g2.at[pl.ds(base, _CHUNK)])
        return c

    lax.fori_loop(0, _CH // 8, body, 0)


def _gather_call():
  return functools.partial(
    pl.kernel,
    mesh=plsc.VectorSubcoreMesh(core_axis_name="c", subcore_axis_name="s", num_cores=_NC, num_subcores=_NS),
    out_type=[
        jax.ShapeDtypeStruct((_EPAD, 2 * _D), jnp.float32),
        jax.ShapeDtypeStruct((_EPAD, _D), jnp.float32),
    ],
    scratch_types=[
        pltpu.VMEM((_IBLK,), jnp.int32),
        pltpu.VMEM((_IBLK,), jnp.int32),
        pltpu.VMEM((_CHUNK, 2 * _D), jnp.float32),
        pltpu.VMEM((_CHUNK, _D), jnp.float32),
        pltpu.SemaphoreType.DMA,
        pltpu.SemaphoreType.DMA,
    ],
  )


def _edge_body(ef, rf, ew, g1, g2,
               k_ef, k_rf, k_hi, k_hj, k_b1, k_g, k_bg, k_w2, k_b2,
               v_ef, v_rf, v_hi, v_hj, v_b1, v_g, v_bg, v_w2, v_b2,
               p16, p16t, m_ref, x_ref):
    hi = g1[:, :_D]
    qd = g1[:, _D:]
    hj = g2[...]
    efb = ef[...]
    rfb = rf[...]

    pre_k = (_mm(efb, k_ef[...]) + _mm(rfb, k_rf[...]) +
             _mm(hi, k_hi[...]) + _mm(hj, k_hj[...]) + k_b1[...])
    k = _mm(_ln_relu(pre_k, k_g[...], k_bg[...]), k_w2[...]) + k_b2[...]

    pre_v = (_mm(efb, v_ef[...]) + _mm(rfb, v_rf[...]) +
             _mm(hi, v_hi[...]) + _mm(hj, v_hj[...]) + v_b1[...])
    v = (_mm(_ln_relu(pre_v, v_g[...], v_bg[...]), v_w2[...]) + v_b2[...]) * ew[...]

    s = _mm(qd * k, p16[...]) * (1.0 / math.sqrt(_HD))
    pos = pl.program_id(0) * _EBLK + lax.broadcasted_iota(jnp.int32, (_EBLK, _H), 0)
    ex = jnp.where(pos < _E, jnp.exp(s), 0.0)
    m_ref[...] = _mm(ex, p16t[...]) * v
    x_ref[:, :_H] = ex
    x_ref[:, _H:] = jnp.zeros((_EBLK, _D - _H), jnp.float32)


def _run_edge(efp, rfp, ewp, g1, g2, pk, pv, p16, p16t):
    full = lambda i: (0, 0)
    eb = lambda i: (i, 0)

    def wspecs():
        return [
            pl.BlockSpec((_EF, _HID), full), pl.BlockSpec((_RF, _HID), full),
            pl.BlockSpec((_D, _HID), full), pl.BlockSpec((_D, _HID), full),
            pl.BlockSpec((1, _HID), full), pl.BlockSpec((1, _HID), full),
            pl.BlockSpec((1, _HID), full), pl.BlockSpec((_HID, _D), full),
            pl.BlockSpec((1, _D), full),
        ]

    def wargs(p):
        return [p['W1'][:_EF], p['W1'][_EF:_EF + _RF],
                p['W1'][_EF + _RF:_EF + _RF + _D], p['W1'][_EF + _RF + _D:],
                p['b1'].reshape(1, -1), p['g'].reshape(1, -1),
                p['bg'].reshape(1, -1), p['W2'], p['b2'].reshape(1, -1)]

    return pl.pallas_call(
        _edge_body,
        grid=(_EPAD // _EBLK,),
        in_specs=[
            pl.BlockSpec((_EBLK, _EF), eb),
            pl.BlockSpec((_EBLK, _RF), eb),
            pl.BlockSpec((_EBLK, 1), eb),
            pl.BlockSpec((_EBLK, 2 * _D), eb),
            pl.BlockSpec((_EBLK, _D), eb),
        ] + wspecs() + wspecs() + [
            pl.BlockSpec((_D, _H), full),
            pl.BlockSpec((_H, _D), full),
        ],
        out_specs=[pl.BlockSpec((_EBLK, _D), eb),
                   pl.BlockSpec((_EBLK, _D), eb)],
        out_shape=[jax.ShapeDtypeStruct((_EPAD, _D), jnp.float32),
                   jax.ShapeDtypeStruct((_EPAD, _D), jnp.float32)],
    )(efp, rfp, ewp, g1, g2, *wargs(pk), *wargs(pv), p16, p16t)


def _rank_body(d_ref, lstrict, offs_ref, base_ref):
    @pl.when(pl.program_id(0) == 0)
    def _():
        base_ref[...] = jnp.zeros((1, _NW), jnp.float32)

    d = d_ref[...]
    tile = d // _CROWS
    tcols = lax.broadcasted_iota(jnp.int32, (_EBLK, _NW), 1)
    onehot = (tile == tcols).astype(jnp.float32)
    prefix = _mm(lstrict[...], onehot)
    base = base_ref[...]
    slotf = jnp.sum(onehot * (prefix + base), -1, keepdims=True)
    slot = jnp.minimum(slotf.astype(jnp.int32), _LCAP - 1)
    offs_ref[...] = tile * _LCAP + slot
    base_ref[...] = base + jnp.sum(onehot, 0, keepdims=True)


def _run_rank(dstp2, lstrict):
    return pl.pallas_call(
        _rank_body,
        grid=(_EPAD // _EBLK,),
        in_specs=[
            pl.BlockSpec((_EBLK, 1), lambda i: (i, 0)),
            pl.BlockSpec((_EBLK, _EBLK), lambda i: (0, 0)),
        ],
        out_specs=pl.BlockSpec((_EBLK, 1), lambda i: (i, 0)),
        out_shape=jax.ShapeDtypeStruct((_EPAD, 1), jnp.int32),
        scratch_shapes=[pltpu.VMEM((1, _NW), jnp.float32)],
    )(dstp2, lstrict)


def _scatter_body(m_hbm, x_hbm, dstp, offs_hbm, fillpos, fillloc, zm, zx,
                  outm, outx, idx_v, off_v, poslist, loclist, mrows, xrows,
                  accm, accx, sem, sem2):
    cid = lax.axis_index("c")
    sid = lax.axis_index("s")
    wid = cid * _NS + sid
    lo = wid * _CROWS
    lane = lax.iota(jnp.int32, 16)

    pltpu.sync_copy(fillpos, poslist)
    pltpu.sync_copy(fillloc, loclist)
    pltpu.sync_copy(zm, accm)
    pltpu.sync_copy(zx, accx)

    def scan(j, dummy):
        base = j * _IBLK
        pltpu.sync_copy(dstp.at[pl.ds(base, _IBLK)], idx_v)
        pltpu.sync_copy(offs_hbm.at[pl.ds(base, _IBLK)], off_v)
        for k in range(_IBLK // 16):
            ix = idx_v[pl.ds(k * 16, 16)]
            lof = off_v[pl.ds(k * 16, 16)] - wid * _LCAP
            ok = jnp.logical_and(lof >= 0, lof < _LCAP)
            lof2 = jnp.where(ok, lof, _LCAP + 16)
            pos16 = base + k * 16 + lane
            plsc.store_scatter(poslist, [lof2], pos16)
            plsc.store_scatter(loclist, [lof2], ix - lo)
        return dummy

    lax.fori_loop(0, _EPAD // _IBLK, scan, 0)

    def batch(b, c):
        bb = b * _BCH
        d1 = pltpu.async_copy(m_hbm.at[poslist.at[pl.ds(bb, _BCH)]], mrows, sem)
        d2 = pltpu.async_copy(x_hbm.at[poslist.at[pl.ds(bb, _BCH)]], xrows,
                              sem2)
        d1.wait()
        d2.wait()

        def acc_grp(gi, c2):
            e0 = gi * 16
            ev = e0 + lane
            loc16 = loclist[pl.ds(bb + e0, 16)]
            for g in range(_D):
                gv = jnp.full((16,), g, jnp.int32)
                vals = plsc.load_gather(mrows, [ev, gv])
                plsc.addupdate_scatter(accm, [loc16, gv], vals)
            for g in range(_H):
                gv = jnp.full((16,), g, jnp.int32)
                vals = plsc.load_gather(xrows, [ev, gv])
                plsc.addupdate_scatter(accx, [loc16, gv], vals)
            return c2

        lax.fori_loop(0, _BCH // 16, acc_grp, 0)
        return c

    lax.fori_loop(0, _LCAP // _BCH, batch, 0)

    pltpu.sync_copy(accm.at[pl.ds(0, _CROWS)], outm.at[pl.ds(lo, _CROWS)])
    pltpu.sync_copy(accx.at[pl.ds(0, _CROWS)], outx.at[pl.ds(lo, _CROWS)])


def _scatter_call():
  return functools.partial(
    pl.kernel,
    mesh=plsc.VectorSubcoreMesh(core_axis_name="c", subcore_axis_name="s", num_cores=_NC, num_subcores=_NS),
    compiler_params=pltpu.CompilerParams(needs_layout_passes=False),
    out_type=[jax.ShapeDtypeStruct((_NOUT, _D), jnp.float32),
              jax.ShapeDtypeStruct((_NOUT, _H), jnp.float32)],
    scratch_types=[
        pltpu.VMEM((_IBLK,), jnp.int32),
        pltpu.VMEM((_IBLK,), jnp.int32),
        pltpu.VMEM((_LCAP + 32,), jnp.int32),
        pltpu.VMEM((_LCAP + 32,), jnp.int32),
        pltpu.VMEM((_BCH, _D), jnp.float32),
        pltpu.VMEM((_BCH, _D), jnp.float32),
        pltpu.VMEM((_CROWS + 8, _D), jnp.float32),
        pltpu.VMEM((_CROWS + 8, _H), jnp.float32),
        pltpu.SemaphoreType.DMA,
        pltpu.SemaphoreType.DMA,
    ],
  )


def _topo_body(m0, x0, h_ref, w1o, w1h, b1, g, bg, w2, b2, p16t, out_ref):
    den = x0[:, :_H] + 1e-16
    r128 = _mm(1.0 / den, p16t[...])
    o = m0[...] * r128
    hb = h_ref[...]
    pre = _mm(o, w1o[...]) + _mm(hb, w1h[...]) + b1[...]
    out_ref[...] = _mm(_ln_relu(pre, g[...], bg[...]), w2[...]) + b2[...]


def _run_topo(pm, px, h, p, p16t):
    full = lambda i: (0, 0)
    nb = lambda i: (i, 0)
    return pl.pallas_call(
        _topo_body,
        grid=(_N // _NBLK,),
        in_specs=[
            pl.BlockSpec((_NBLK, _D), nb),
            pl.BlockSpec((_NBLK, _D), nb),
            pl.BlockSpec((_NBLK, _D), nb),
            pl.BlockSpec((_D, _HID), full), pl.BlockSpec((_D, _HID), full),
            pl.BlockSpec((1, _HID), full), pl.BlockSpec((1, _HID), full),
            pl.BlockSpec((1, _HID), full), pl.BlockSpec((_HID, _HID), full),
            pl.BlockSpec((1, _HID), full), pl.BlockSpec((_H, _D), full),
        ],
        out_specs=pl.BlockSpec((_NBLK, _HID), nb),
        out_shape=jax.ShapeDtypeStruct((_N, _HID), jnp.float32),
    )(pm, px, h,
      p['W1'][:_D], p['W1'][_D:], p['b1'].reshape(1, -1),
      p['g'].reshape(1, -1), p['bg'].reshape(1, -1), p['W2'],
      p['b2'].reshape(1, -1), p16t)


def kernel(h, r_feat, edge_feat, e_w, params, edge_index):
    src = edge_index[0]
    dst = edge_index[1]
    pad = _EPAD - _E
    dstp = jnp.pad(dst, (0, pad))
    srcp = jnp.pad(src, (0, pad))
    efp = jnp.pad(edge_feat, ((0, pad), (0, 0)))
    rfp = jnp.pad(r_feat, ((0, pad), (0, 0)))
    ewp = jnp.pad(e_w, (0, pad)).reshape(_EPAD, 1)

    p16 = jnp.asarray(np.repeat(np.eye(_H, dtype=np.float32), _HD, axis=0))
    p16t = jnp.asarray(p16.T)

    t1 = _run_q(h, params['xq'])
    g1, g2 = _gather_call()(_gather_body)(t1, h, dstp, srcp)
    m, x = _run_edge(efp, rfp, ewp, g1, g2, params['xk'], params['xv'],
                     p16, p16t)
    fillpos = jnp.full((_LCAP + 32,), _E, jnp.int32)
    fillloc = jnp.full((_LCAP + 32,), _CROWS, jnp.int32)
    zm = jnp.zeros((_CROWS + 8, _D), jnp.float32)
    zx = jnp.zeros((_CROWS + 8, _H), jnp.float32)
    dsts = jnp.pad(dst, (0, pad), constant_values=_N + 16)
    lstrict = jnp.asarray(np.tril(np.ones((_EBLK, _EBLK), np.float32), -1))
    offs = _run_rank(dsts.reshape(_EPAD, 1), lstrict).reshape(_EPAD)
    pm, px = _scatter_call()(_scatter_body)(m, x, dsts, offs, fillpos, fillloc,
                                            zm, zx)
    return _run_topo(pm, px, h, params['topo'], p16t)
